# baseline (speedup 1.0000x reference)
import functools

import jax
import jax.numpy as jnp
from jax import lax
from jax.experimental import pallas as pl
from jax.experimental.pallas import tpu as pltpu
from jax.experimental.pallas import tpu_sc as plsc

BR = 2048
RS = 16
S_STATIC = 8192

_SC_NW = 16
_CHUNK = 128


def _mask_scatter_kernel(T, C):
    rows_per_w = T // _SC_NW
    mesh = plsc.VectorSubcoreMesh(
        core_axis_name="c", subcore_axis_name="s", num_cores=1)

    @functools.partial(
        pl.kernel, mesh=mesh,
        out_type=jax.ShapeDtypeStruct((T,), jnp.float32),
        scratch_types=[
            pltpu.VMEM((C, _CHUNK), jnp.int32),
            pltpu.VMEM((_CHUNK,), jnp.float32),
            pltpu.VMEM((rows_per_w,), jnp.float32),
            pltpu.SemaphoreType.DMA,
        ],
    )
    def k(idx_hbm, mask_hbm, idx_v, ones_v, zeros_v, sem):
        w = lax.axis_index("s")
        for j in range(rows_per_w // 16):
            zeros_v[pl.ds(j * 16, 16)] = jnp.zeros((16,), jnp.float32)
        for j in range(_CHUNK // 16):
            ones_v[pl.ds(j * 16, 16)] = jnp.ones((16,), jnp.float32)
        pltpu.sync_copy(zeros_v, mask_hbm.at[pl.ds(w * rows_per_w, rows_per_w)])
        plsc.subcore_barrier()
        pltpu.sync_copy(idx_hbm.at[w], idx_v)
        for j in range(C):
            pltpu.async_copy(ones_v, mask_hbm.at[idx_v.at[j]], sem).wait()

    return k


def _fused_body(x_ref, mask_ref, o_ref, carry_ref):
    i = pl.program_id(0)
    c0 = carry_ref[...]
    c0 = jnp.where((i * BR) % S_STATIC == 0, jnp.zeros_like(c0), c0)

    prev = c0
    for s in range(BR // RS):
        base = s * RS
        rows = x_ref[base:base + RS]
        ext = jnp.concatenate([prev[5:], rows], axis=0)
        m2 = jnp.minimum(ext[1:], ext[:-1])
        m = jnp.minimum(m2[2:], m2[:RS])
        sig = jax.nn.sigmoid(rows)
        w = mask_ref[base:base + RS] * sig
        o_ref[base:base + RS] = rows + w * (m - rows)
        prev = rows[RS - 8:]

    carry_ref[...] = prev


def kernel(_x, rand_index):
    B, S, H = _x.shape
    assert S == S_STATIC and S % BR == 0
    T = B * S
    K = rand_index.shape[0]

    per_w = _SC_NW * _CHUNK
    C = -(-K // per_w)
    KP = _SC_NW * C * _CHUNK
    idx = jnp.concatenate(
        [rand_index, jnp.broadcast_to(rand_index[-1], (KP - K,))])
    idx = idx.reshape(_SC_NW, C, _CHUNK)

    mask = _mask_scatter_kernel(T, C)(idx)
    mask = mask.reshape(T, 1)

    xf = _x.reshape(T, H)
    out = pl.pallas_call(
        _fused_body,
        grid=(T // BR,),
        in_specs=[
            pl.BlockSpec((BR, H), lambda i: (i, 0)),
            pl.BlockSpec((BR, 1), lambda i: (i, 0)),
        ],
        out_specs=pl.BlockSpec((BR, H), lambda i: (i, 0)),
        out_shape=jax.ShapeDtypeStruct((T, H), jnp.float32),
        scratch_shapes=[pltpu.VMEM((8, H), jnp.float32)],
    )(xf, mask)
    return out.reshape(B, S, H)


# speedup vs baseline: 1.9514x; 1.9514x over previous
import functools

import jax
import jax.numpy as jnp
from jax import lax
from jax.experimental import pallas as pl
from jax.experimental.pallas import tpu as pltpu
from jax.experimental.pallas import tpu_sc as plsc

BR = 2048
RS = 8
S_STATIC = 8192

_SC_NW = 16
_CHUNK = 128


def _mask_scatter_kernel(T, C):
    rows_per_w = T // _SC_NW
    mesh = plsc.VectorSubcoreMesh(
        core_axis_name="c", subcore_axis_name="s", num_cores=1)

    @functools.partial(
        pl.kernel, mesh=mesh,
        out_type=jax.ShapeDtypeStruct((T,), jnp.float32),
        scratch_types=[
            pltpu.VMEM((C, _CHUNK), jnp.int32),
            pltpu.VMEM((_CHUNK,), jnp.float32),
            pltpu.VMEM((rows_per_w,), jnp.float32),
            pltpu.SemaphoreType.DMA,
        ],
    )
    def k(idx_hbm, mask_hbm, idx_v, ones_v, zeros_v, sem):
        w = lax.axis_index("s")
        for j in range(rows_per_w // 16):
            zeros_v[pl.ds(j * 16, 16)] = jnp.zeros((16,), jnp.float32)
        for j in range(_CHUNK // 16):
            ones_v[pl.ds(j * 16, 16)] = jnp.ones((16,), jnp.float32)
        pltpu.sync_copy(zeros_v, mask_hbm.at[pl.ds(w * rows_per_w, rows_per_w)])
        plsc.subcore_barrier()
        pltpu.sync_copy(idx_hbm.at[w], idx_v)
        for j in range(C):
            pltpu.async_copy(ones_v, mask_hbm.at[idx_v.at[j]], sem).wait()

    return k


def _fused_body(x_ref, mask_ref, o_ref, carry_ref):
    i = pl.program_id(0)
    c0 = carry_ref[...]
    c0 = jnp.where((i * BR) % S_STATIC == 0, jnp.zeros_like(c0), c0)

    prev = c0
    for s in range(BR // RS):
        base = s * RS
        rows = x_ref[base:base + RS]
        ext = jnp.concatenate([prev[5:], rows], axis=0)
        m2 = jnp.minimum(ext[1:], ext[:-1])
        m = jnp.minimum(m2[2:], m2[:RS])
        sig = jax.nn.sigmoid(rows)
        w = mask_ref[base:base + RS] * sig
        o_ref[base:base + RS] = rows + w * (m - rows)
        prev = rows[RS - 8:]

    carry_ref[...] = prev


def kernel(_x, rand_index):
    B, S, H = _x.shape
    assert S == S_STATIC and S % BR == 0
    T = B * S
    K = rand_index.shape[0]

    per_w = _SC_NW * _CHUNK
    C = -(-K // per_w)
    KP = _SC_NW * C * _CHUNK
    idx = jnp.concatenate(
        [rand_index, jnp.broadcast_to(rand_index[-1], (KP - K,))])
    idx = idx.reshape(_SC_NW, C, _CHUNK)

    mask = jnp.zeros((T, 1), jnp.float32).at[rand_index].set(1.0)
    del idx, C

    xf = _x.reshape(T, H)
    out = pl.pallas_call(
        _fused_body,
        grid=(T // BR,),
        in_specs=[
            pl.BlockSpec((BR, H), lambda i: (i, 0)),
            pl.BlockSpec((BR, 1), lambda i: (i, 0)),
        ],
        out_specs=pl.BlockSpec((BR, H), lambda i: (i, 0)),
        out_shape=jax.ShapeDtypeStruct((T, H), jnp.float32),
        scratch_shapes=[pltpu.VMEM((8, H), jnp.float32)],
    )(xf, mask)
    return out.reshape(B, S, H)


# speedup vs baseline: 1.9597x; 1.0043x over previous
import functools

import jax
import jax.numpy as jnp
from jax import lax
from jax.experimental import pallas as pl
from jax.experimental.pallas import tpu as pltpu
from jax.experimental.pallas import tpu_sc as plsc

BR = 2048
RS = 16
HB = 512
S_STATIC = 8192

_SC_NW = 16
_CHUNK = 128


def _mask_scatter_kernel(T, C):
    rows_per_w = T // _SC_NW
    mesh = plsc.VectorSubcoreMesh(
        core_axis_name="c", subcore_axis_name="s", num_cores=1)

    @functools.partial(
        pl.kernel, mesh=mesh,
        out_type=jax.ShapeDtypeStruct((T,), jnp.float32),
        scratch_types=[
            pltpu.VMEM((C, _CHUNK), jnp.int32),
            pltpu.VMEM((_CHUNK,), jnp.float32),
            pltpu.VMEM((rows_per_w,), jnp.float32),
            pltpu.SemaphoreType.DMA,
        ],
    )
    def k(idx_hbm, mask_hbm, idx_v, ones_v, zeros_v, sem):
        w = lax.axis_index("s")
        for j in range(rows_per_w // 16):
            zeros_v[pl.ds(j * 16, 16)] = jnp.zeros((16,), jnp.float32)
        for j in range(_CHUNK // 16):
            ones_v[pl.ds(j * 16, 16)] = jnp.ones((16,), jnp.float32)
        pltpu.sync_copy(zeros_v, mask_hbm.at[pl.ds(w * rows_per_w, rows_per_w)])
        plsc.subcore_barrier()
        pltpu.sync_copy(idx_hbm.at[w], idx_v)
        for j in range(C):
            pltpu.async_copy(ones_v, mask_hbm.at[idx_v.at[j]], sem).wait()

    return k


def _fused_body(x_ref, mask_ref, o_ref, carry_ref):
    i = pl.program_id(0)
    j = pl.program_id(1)
    c0 = carry_ref[:, pl.ds(j * HB, HB)]
    c0 = jnp.where((i * BR) % S_STATIC == 0, jnp.zeros_like(c0), c0)

    prev = c0
    for s in range(BR // RS):
        base = s * RS
        rows = x_ref[base:base + RS]
        ext = jnp.concatenate([prev[5:], rows], axis=0)
        m2 = jnp.minimum(ext[1:], ext[:-1])
        m = jnp.minimum(m2[2:], m2[:RS])
        sig = jax.nn.sigmoid(rows)
        w = mask_ref[base:base + RS] * sig
        o_ref[base:base + RS] = rows + w * (m - rows)
        prev = rows[RS - 8:]

    carry_ref[:, pl.ds(j * HB, HB)] = prev


def kernel(_x, rand_index):
    B, S, H = _x.shape
    assert S == S_STATIC and S % BR == 0
    T = B * S
    K = rand_index.shape[0]

    per_w = _SC_NW * _CHUNK
    C = -(-K // per_w)
    KP = _SC_NW * C * _CHUNK
    idx = jnp.concatenate(
        [rand_index, jnp.broadcast_to(rand_index[-1], (KP - K,))])
    idx = idx.reshape(_SC_NW, C, _CHUNK)

    mask = jnp.zeros((T, 1), jnp.float32).at[rand_index].set(1.0)
    del idx, C

    xf = _x.reshape(T, H)
    out = pl.pallas_call(
        _fused_body,
        grid=(T // BR, H // HB),
        in_specs=[
            pl.BlockSpec((BR, HB), lambda i, j: (i, j)),
            pl.BlockSpec((BR, 1), lambda i, j: (i, 0)),
        ],
        out_specs=pl.BlockSpec((BR, HB), lambda i, j: (i, j)),
        out_shape=jax.ShapeDtypeStruct((T, H), jnp.float32),
        scratch_shapes=[pltpu.VMEM((8, H), jnp.float32)],
    )(xf, mask)
    return out.reshape(B, S, H)


# speedup vs baseline: 1.9905x; 1.0157x over previous
import jax
import jax.numpy as jnp
from jax.experimental import pallas as pl
from jax.experimental.pallas import tpu as pltpu

BR = 2048
RS = 16
S_STATIC = 8192


def _fused_body(x_ref, mask_ref, o_ref, carry_ref):
    i = pl.program_id(0)
    c0 = carry_ref[...]
    c0 = jnp.where((i * BR) % S_STATIC == 0, jnp.zeros_like(c0), c0)

    prev = c0
    for s in range(BR // RS):
        base = s * RS
        rows = x_ref[base:base + RS]
        ext = jnp.concatenate([prev[5:], rows], axis=0)
        m2 = jnp.minimum(ext[1:], ext[:-1])
        m = jnp.minimum(m2[2:], m2[:RS])
        sig = jax.nn.sigmoid(rows)
        w = mask_ref[base:base + RS] * sig
        o_ref[base:base + RS] = rows + w * (m - rows)
        prev = rows[RS - 8:]

    carry_ref[...] = prev


def kernel(_x, rand_index):
    B, S, H = _x.shape
    assert S == S_STATIC and S % BR == 0
    T = B * S
    mask = jnp.zeros((T, 1), jnp.float32).at[rand_index].set(1.0)

    xf = _x.reshape(T, H)
    out = pl.pallas_call(
        _fused_body,
        grid=(T // BR,),
        in_specs=[
            pl.BlockSpec((BR, H), lambda i: (i, 0)),
            pl.BlockSpec((BR, 1), lambda i: (i, 0)),
        ],
        out_specs=pl.BlockSpec((BR, H), lambda i: (i, 0)),
        out_shape=jax.ShapeDtypeStruct((T, H), jnp.float32),
        scratch_shapes=[pltpu.VMEM((8, H), jnp.float32)],
    )(xf, mask)
    return out.reshape(B, S, H)
